# baseline (speedup 1.0000x reference)
import jax
import jax.numpy as jnp
from jax import lax
from jax.experimental import pallas as pl
from jax.experimental.pallas import tpu as pltpu
from jax.experimental.pallas import tpu_sc as plsc

NUM_EMB = 1_000_000
DIM = 32
PAD_IDX = 1
B = 4096
L = 200

NC = 2
NS = 16
NW = NC * NS

BLK = 128
N_FULL_BLK = NUM_EMB // BLK
TAIL = NUM_EMB - N_FULL_BLK * BLK
BLK_PER_W = N_FULL_BLK // NW
N_EXTRA = N_FULL_BLK - BLK_PER_W * NW

ROWS_PER_W = B // NW
CHUNK = 4
N_CHUNKS = ROWS_PER_W // CHUNK
IDX_PER_CHUNK = CHUNK * L
G_SPLIT = (104, 96)
NBUF = 3


def _transpose_block(ibuf, obuf, iota16, ncols):
    for fg in (0, 1):
        f0 = 16 * fg
        frow = f0 + iota16
        rots = [(d + iota16) & 15 for d in range(16)]
        ocols = [(((d + iota16) & 15 & 3) << 5) + frow for d in range(16)]

        def cb_body(cb, _):
            c0 = cb * 16
            c4 = cb * 4
            for d in range(16):
                col = c0 + rots[d]
                v = plsc.load_gather(ibuf, [frow, col])
                orow = c4 + (rots[d] >> 2)
                plsc.store_scatter(obuf, [orow, ocols[d]], v)
            return 0

        lax.fori_loop(0, ncols // 16, cb_body, 0)


def _k1_body(tT_hbm, out_hbm, ib0, ib1, ob0, ob1, itail, otail,
             rs0, rs1, ws0, ws1):
    wid = lax.axis_index("s") * NC + lax.axis_index("c")
    iota16 = lax.iota(jnp.int32, 16)
    ibufs, obufs = (ib0, ib1), (ob0, ob1)
    rsems, wsems = (rs0, rs1), (ws0, ws1)

    def blk_of(t):
        return wid * BLK_PER_W + t

    def fire_read(t, p):
        col0 = pl.multiple_of(blk_of(t) * BLK, BLK)
        pltpu.async_copy(tT_hbm.at[:, pl.ds(col0, BLK)], ibufs[p], rsems[p])

    def wait_read(p):
        pltpu.make_async_copy(tT_hbm.at[:, pl.ds(0, BLK)], ibufs[p],
                              rsems[p]).wait()

    def fire_write(t, p):
        row0 = pl.multiple_of(blk_of(t) * (BLK // 4), BLK // 4)
        pltpu.async_copy(obufs[p], out_hbm.at[pl.ds(row0, BLK // 4)], wsems[p])

    def wait_write(p):
        pltpu.make_async_copy(obufs[p], out_hbm.at[pl.ds(0, BLK // 4)],
                              wsems[p]).wait()

    fire_read(0, 0)
    fire_read(1, 1)

    def outer(ko, _):
        for p in (0, 1):
            t = 2 * ko + p
            wait_read(p)

            @pl.when(ko > 0)
            def _():
                wait_write(p)
            _transpose_block(ibufs[p], obufs[p], iota16, BLK)

            @pl.when(t + 2 < BLK_PER_W)
            def _():
                fire_read(t + 2, p)
            fire_write(t, p)
        return 0

    lax.fori_loop(0, BLK_PER_W // 2, outer, 0)
    wait_write(0)
    wait_write(1)

    @pl.when(wid < N_EXTRA)
    def _():
        col0 = pl.multiple_of((NW * BLK_PER_W + wid) * BLK, BLK)
        row0 = pl.multiple_of((NW * BLK_PER_W + wid) * (BLK // 4), BLK // 4)
        pltpu.sync_copy(tT_hbm.at[:, pl.ds(col0, BLK)], ibufs[0])
        _transpose_block(ibufs[0], obufs[0], iota16, BLK)
        pltpu.sync_copy(obufs[0], out_hbm.at[pl.ds(row0, BLK // 4)])

    @pl.when(wid == N_EXTRA)
    def _():
        col0 = N_FULL_BLK * BLK
        pltpu.sync_copy(tT_hbm.at[:, pl.ds(col0, TAIL)], itail)
        _transpose_block(itail, otail, iota16, TAIL)
        pltpu.sync_copy(otail, out_hbm.at[pl.ds(N_FULL_BLK * (BLK // 4), TAIL // 4)])


def _k2_body(x_hbm, table_hbm, out_hbm,
             idx0, idx1, idx2, rows0, rows1, rows2, out_v, pad_v,
             is0, is1, is2, rs0, rs1, rs2):
    wid = lax.axis_index("s") * NC + lax.axis_index("c")

    pltpu.sync_copy(table_hbm.at[pl.ds(PAD_IDX, 1)], pad_v)
    t1_lo = pad_v[0, pl.ds(0, 16)]
    t1_hi = pad_v[0, pl.ds(16, 16)]
    inv_l = jnp.float32(1.0 / L)
    tail_mask = lax.iota(jnp.int32, 16) >= 8

    idxs = (idx0, idx1, idx2)
    rows = (rows0, rows1, rows2)
    isems = (is0, is1, is2)
    rsems = (rs0, rs1, rs2)

    def fire_idx(k, p):
        pltpu.async_copy(
            x_hbm.at[pl.ds((wid * N_CHUNKS + k) * CHUNK, CHUNK)],
            idxs[p], isems[p])

    def wait_idx(p):
        pltpu.make_async_copy(
            x_hbm.at[pl.ds(0, CHUNK)], idxs[p], isems[p]).wait()

    def fire_gathers(p):
        for b in range(CHUNK):
            off = 0
            for g in G_SPLIT:
                pltpu.async_copy(
                    table_hbm.at[idxs[p].at[b, pl.ds(off, g)]],
                    rows[p].at[pl.ds(b * L + off, g)],
                    rsems[p],
                )
                off += g

    def drain_rows(p):
        pltpu.make_async_copy(
            table_hbm.at[pl.ds(0, IDX_PER_CHUNK)], rows[p], rsems[p]
        ).wait()

    def accumulate(k, p):
        idx_v, rows_v = idxs[p], rows[p]
        for b in range(CHUNK):
            cnt = jnp.zeros((16,), jnp.float32)
            for t in range(12):
                iv = idx_v[b, pl.ds(t * 16, 16)]
                cnt = cnt + jnp.where(iv == PAD_IDX, 1.0, 0.0)
            ivt = idx_v[b, pl.ds(L - 16, 16)]
            cnt = cnt + jnp.where((ivt == PAD_IDX) & tail_mask, 1.0, 0.0)
            npad = jnp.sum(cnt)

            def row_body(t, accs):
                base = b * L + t * 16
                accs = list(accs)
                for u in range(16):
                    fr = base + u
                    accs[u % 4] = accs[u % 4] + rows_v[fr, pl.ds(0, 16)]
                    accs[4 + u % 4] = accs[4 + u % 4] + rows_v[fr, pl.ds(16, 16)]
                return tuple(accs)

            zero = jnp.zeros((16,), jnp.float32)
            accs = list(lax.fori_loop(0, 12, row_body, (zero,) * 8))
            for u in range(8):
                fr = b * L + 192 + u
                accs[u % 4] = accs[u % 4] + rows_v[fr, pl.ds(0, 16)]
                accs[4 + u % 4] = accs[4 + u % 4] + rows_v[fr, pl.ds(16, 16)]
            s_lo = (accs[0] + accs[1]) + (accs[2] + accs[3])
            s_hi = (accs[4] + accs[5]) + (accs[6] + accs[7])
            row = k * CHUNK + b
            out_v[row, pl.ds(0, 16)] = (s_lo - npad * t1_lo) * inv_l
            out_v[row, pl.ds(16, 16)] = (s_hi - npad * t1_hi) * inv_l

    fire_idx(0, 0)
    fire_idx(1, 1)
    fire_idx(2, 2)
    wait_idx(0)
    fire_gathers(0)
    wait_idx(1)
    fire_gathers(1)

    N_TRIPLES = N_CHUNKS // 3

    def outer(ko, _):
        for j in range(3):
            k = 3 * ko + j
            p = j
            pn2 = (j + 2) % 3
            wait_idx(pn2)
            fire_gathers(pn2)
            drain_rows(p)
            accumulate(k, p)
            if j < 2:
                fire_idx(k + 3, p)
            else:
                @pl.when(ko < N_TRIPLES - 1)
                def _():
                    fire_idx(k + 3, p)
        return 0

    lax.fori_loop(0, N_TRIPLES, outer, 0)
    for k, p in ((N_CHUNKS - 2, 0), (N_CHUNKS - 1, 1)):
        drain_rows(p)
        accumulate(k, p)

    pltpu.sync_copy(out_v, out_hbm.at[pl.ds(wid * ROWS_PER_W, ROWS_PER_W)])


@jax.jit
def kernel(x, table):
    mesh = plsc.VectorSubcoreMesh(
        core_axis_name="c", subcore_axis_name="s", num_cores=NC, num_subcores=NS
    )

    k1 = pl.kernel(
        _k1_body,
        out_type=jax.ShapeDtypeStruct((NUM_EMB // 4, 4 * DIM), jnp.float32),
        mesh=mesh,
        compiler_params=pltpu.CompilerParams(
            needs_layout_passes=False, use_tc_tiling_on_sc=True
        ),
        scratch_types=[
            pltpu.VMEM((DIM, BLK), jnp.float32),
            pltpu.VMEM((DIM, BLK), jnp.float32),
            pltpu.VMEM((BLK // 4, 4 * DIM), jnp.float32),
            pltpu.VMEM((BLK // 4, 4 * DIM), jnp.float32),
            pltpu.VMEM((DIM, TAIL), jnp.float32),
            pltpu.VMEM((TAIL // 4, 4 * DIM), jnp.float32),
            pltpu.SemaphoreType.DMA,
            pltpu.SemaphoreType.DMA,
            pltpu.SemaphoreType.DMA,
            pltpu.SemaphoreType.DMA,
        ],
    )
    table_lin = k1(table.T).reshape(NUM_EMB, DIM)

    k2 = pl.kernel(
        _k2_body,
        out_type=jax.ShapeDtypeStruct((B, DIM), jnp.float32),
        mesh=mesh,
        compiler_params=pltpu.CompilerParams(
            needs_layout_passes=False, use_tc_tiling_on_sc=False
        ),
        scratch_types=[
            pltpu.VMEM((CHUNK, L), jnp.int32),
            pltpu.VMEM((CHUNK, L), jnp.int32),
            pltpu.VMEM((CHUNK, L), jnp.int32),
            pltpu.VMEM((IDX_PER_CHUNK, DIM), jnp.float32),
            pltpu.VMEM((IDX_PER_CHUNK, DIM), jnp.float32),
            pltpu.VMEM((IDX_PER_CHUNK, DIM), jnp.float32),
            pltpu.VMEM((ROWS_PER_W, DIM), jnp.float32),
            pltpu.VMEM((1, DIM), jnp.float32),
            pltpu.SemaphoreType.DMA,
            pltpu.SemaphoreType.DMA,
            pltpu.SemaphoreType.DMA,
            pltpu.SemaphoreType.DMA,
            pltpu.SemaphoreType.DMA,
            pltpu.SemaphoreType.DMA,
        ],
    )
    return k2(x, table_lin)


# speedup vs baseline: 1.2499x; 1.2499x over previous
import jax
import jax.numpy as jnp
from jax import lax
from jax.experimental import pallas as pl
from jax.experimental.pallas import tpu as pltpu
from jax.experimental.pallas import tpu_sc as plsc

NUM_EMB = 1_000_000
DIM = 32
PAD_IDX = 1
B = 4096
L = 200

NC = 2
NS = 16
NW = NC * NS

BLK = 128
N_FULL_BLK = NUM_EMB // BLK
TAIL = NUM_EMB - N_FULL_BLK * BLK
BLK_PER_W = N_FULL_BLK // NW
N_EXTRA = N_FULL_BLK - BLK_PER_W * NW

ROWS_PER_W = B // NW
CHUNK = 4
N_CHUNKS = ROWS_PER_W // CHUNK
IDX_PER_CHUNK = CHUNK * L
G_SPLIT = (104, 96)
NBUF = 3


def _transpose_block(ibuf, obuf, iota16, ncols):
    for fg in (0, 1):
        f0 = 16 * fg
        frow = f0 + iota16

        @plsc.parallel_loop(0, ncols // 16, unroll=2)
        def cb_body(cb):
            c0 = cb * 16
            c4 = cb * 4
            for d in range(16):
                rot = (d + iota16) & 15
                v = plsc.load_gather(ibuf, [frow, c0 + rot])
                orow = c4 + (rot >> 2)
                ocol = ((rot & 3) << 5) + frow
                plsc.store_scatter(obuf, [orow, ocol], v)


def _k1_body(tT_hbm, out_hbm, ib0, ib1, ob0, ob1, itail, otail,
             rs0, rs1, ws0, ws1):
    wid = lax.axis_index("s") * NC + lax.axis_index("c")
    iota16 = lax.iota(jnp.int32, 16)
    ibufs, obufs = (ib0, ib1), (ob0, ob1)
    rsems, wsems = (rs0, rs1), (ws0, ws1)

    def blk_of(t):
        return wid * BLK_PER_W + t

    def fire_read(t, p):
        col0 = pl.multiple_of(blk_of(t) * BLK, BLK)
        pltpu.async_copy(tT_hbm.at[:, pl.ds(col0, BLK)], ibufs[p], rsems[p])

    def wait_read(p):
        pltpu.make_async_copy(tT_hbm.at[:, pl.ds(0, BLK)], ibufs[p],
                              rsems[p]).wait()

    def fire_write(t, p):
        row0 = pl.multiple_of(blk_of(t) * (BLK // 4), BLK // 4)
        pltpu.async_copy(obufs[p], out_hbm.at[pl.ds(row0, BLK // 4)], wsems[p])

    def wait_write(p):
        pltpu.make_async_copy(obufs[p], out_hbm.at[pl.ds(0, BLK // 4)],
                              wsems[p]).wait()

    fire_read(0, 0)
    fire_read(1, 1)

    def outer(ko, _):
        for p in (0, 1):
            t = 2 * ko + p
            wait_read(p)

            @pl.when(ko > 0)
            def _():
                wait_write(p)
            _transpose_block(ibufs[p], obufs[p], iota16, BLK)

            @pl.when(t + 2 < BLK_PER_W)
            def _():
                fire_read(t + 2, p)
            fire_write(t, p)
        return 0

    lax.fori_loop(0, BLK_PER_W // 2, outer, 0)
    wait_write(0)
    wait_write(1)

    @pl.when(wid < N_EXTRA)
    def _():
        col0 = pl.multiple_of((NW * BLK_PER_W + wid) * BLK, BLK)
        row0 = pl.multiple_of((NW * BLK_PER_W + wid) * (BLK // 4), BLK // 4)
        pltpu.sync_copy(tT_hbm.at[:, pl.ds(col0, BLK)], ibufs[0])
        _transpose_block(ibufs[0], obufs[0], iota16, BLK)
        pltpu.sync_copy(obufs[0], out_hbm.at[pl.ds(row0, BLK // 4)])

    @pl.when(wid == N_EXTRA)
    def _():
        col0 = N_FULL_BLK * BLK
        pltpu.sync_copy(tT_hbm.at[:, pl.ds(col0, TAIL)], itail)
        _transpose_block(itail, otail, iota16, TAIL)
        pltpu.sync_copy(otail, out_hbm.at[pl.ds(N_FULL_BLK * (BLK // 4), TAIL // 4)])


def _k2_body(x_hbm, table_hbm, out_hbm,
             idx0, idx1, idx2, rows0, rows1, rows2, out_v, pad_v,
             is0, is1, is2, rs0, rs1, rs2):
    wid = lax.axis_index("s") * NC + lax.axis_index("c")

    pltpu.sync_copy(table_hbm.at[pl.ds(PAD_IDX, 1)], pad_v)
    t1_lo = pad_v[0, pl.ds(0, 16)]
    t1_hi = pad_v[0, pl.ds(16, 16)]
    inv_l = jnp.float32(1.0 / L)
    tail_mask = lax.iota(jnp.int32, 16) >= 8

    idxs = (idx0, idx1, idx2)
    rows = (rows0, rows1, rows2)
    isems = (is0, is1, is2)
    rsems = (rs0, rs1, rs2)

    def fire_idx(k, p):
        pltpu.async_copy(
            x_hbm.at[pl.ds((wid * N_CHUNKS + k) * CHUNK, CHUNK)],
            idxs[p], isems[p])

    def wait_idx(p):
        pltpu.make_async_copy(
            x_hbm.at[pl.ds(0, CHUNK)], idxs[p], isems[p]).wait()

    def fire_gathers(p):
        for b in range(CHUNK):
            off = 0
            for g in G_SPLIT:
                pltpu.async_copy(
                    table_hbm.at[idxs[p].at[b, pl.ds(off, g)]],
                    rows[p].at[pl.ds(b * L + off, g)],
                    rsems[p],
                )
                off += g

    def drain_rows(p):
        pltpu.make_async_copy(
            table_hbm.at[pl.ds(0, IDX_PER_CHUNK)], rows[p], rsems[p]
        ).wait()

    def accumulate(k, p):
        idx_v, rows_v = idxs[p], rows[p]
        for b in range(CHUNK):
            cnt = jnp.zeros((16,), jnp.float32)
            for t in range(12):
                iv = idx_v[b, pl.ds(t * 16, 16)]
                cnt = cnt + jnp.where(iv == PAD_IDX, 1.0, 0.0)
            ivt = idx_v[b, pl.ds(L - 16, 16)]
            cnt = cnt + jnp.where((ivt == PAD_IDX) & tail_mask, 1.0, 0.0)
            npad = jnp.sum(cnt)

            def row_body(t, accs):
                base = b * L + t * 16
                accs = list(accs)
                for u in range(16):
                    fr = base + u
                    accs[u % 4] = accs[u % 4] + rows_v[fr, pl.ds(0, 16)]
                    accs[4 + u % 4] = accs[4 + u % 4] + rows_v[fr, pl.ds(16, 16)]
                return tuple(accs)

            zero = jnp.zeros((16,), jnp.float32)
            accs = list(lax.fori_loop(0, 12, row_body, (zero,) * 8))
            for u in range(8):
                fr = b * L + 192 + u
                accs[u % 4] = accs[u % 4] + rows_v[fr, pl.ds(0, 16)]
                accs[4 + u % 4] = accs[4 + u % 4] + rows_v[fr, pl.ds(16, 16)]
            s_lo = (accs[0] + accs[1]) + (accs[2] + accs[3])
            s_hi = (accs[4] + accs[5]) + (accs[6] + accs[7])
            row = k * CHUNK + b
            out_v[row, pl.ds(0, 16)] = (s_lo - npad * t1_lo) * inv_l
            out_v[row, pl.ds(16, 16)] = (s_hi - npad * t1_hi) * inv_l

    fire_idx(0, 0)
    fire_idx(1, 1)
    fire_idx(2, 2)
    wait_idx(0)
    fire_gathers(0)
    wait_idx(1)
    fire_gathers(1)

    N_TRIPLES = N_CHUNKS // 3

    def outer(ko, _):
        for j in range(3):
            k = 3 * ko + j
            p = j
            pn2 = (j + 2) % 3
            wait_idx(pn2)
            fire_gathers(pn2)
            drain_rows(p)
            accumulate(k, p)
            if j < 2:
                fire_idx(k + 3, p)
            else:
                @pl.when(ko < N_TRIPLES - 1)
                def _():
                    fire_idx(k + 3, p)
        return 0

    lax.fori_loop(0, N_TRIPLES, outer, 0)
    for k, p in ((N_CHUNKS - 2, 0), (N_CHUNKS - 1, 1)):
        drain_rows(p)
        accumulate(k, p)

    pltpu.sync_copy(out_v, out_hbm.at[pl.ds(wid * ROWS_PER_W, ROWS_PER_W)])


@jax.jit
def kernel(x, table):
    mesh = plsc.VectorSubcoreMesh(
        core_axis_name="c", subcore_axis_name="s", num_cores=NC, num_subcores=NS
    )

    k1 = pl.kernel(
        _k1_body,
        out_type=jax.ShapeDtypeStruct((NUM_EMB // 4, 4 * DIM), jnp.float32),
        mesh=mesh,
        compiler_params=pltpu.CompilerParams(
            needs_layout_passes=False, use_tc_tiling_on_sc=True
        ),
        scratch_types=[
            pltpu.VMEM((DIM, BLK), jnp.float32),
            pltpu.VMEM((DIM, BLK), jnp.float32),
            pltpu.VMEM((BLK // 4, 4 * DIM), jnp.float32),
            pltpu.VMEM((BLK // 4, 4 * DIM), jnp.float32),
            pltpu.VMEM((DIM, TAIL), jnp.float32),
            pltpu.VMEM((TAIL // 4, 4 * DIM), jnp.float32),
            pltpu.SemaphoreType.DMA,
            pltpu.SemaphoreType.DMA,
            pltpu.SemaphoreType.DMA,
            pltpu.SemaphoreType.DMA,
        ],
    )
    table_lin = k1(table.T).reshape(NUM_EMB, DIM)

    k2 = pl.kernel(
        _k2_body,
        out_type=jax.ShapeDtypeStruct((B, DIM), jnp.float32),
        mesh=mesh,
        compiler_params=pltpu.CompilerParams(
            needs_layout_passes=False, use_tc_tiling_on_sc=False
        ),
        scratch_types=[
            pltpu.VMEM((CHUNK, L), jnp.int32),
            pltpu.VMEM((CHUNK, L), jnp.int32),
            pltpu.VMEM((CHUNK, L), jnp.int32),
            pltpu.VMEM((IDX_PER_CHUNK, DIM), jnp.float32),
            pltpu.VMEM((IDX_PER_CHUNK, DIM), jnp.float32),
            pltpu.VMEM((IDX_PER_CHUNK, DIM), jnp.float32),
            pltpu.VMEM((ROWS_PER_W, DIM), jnp.float32),
            pltpu.VMEM((1, DIM), jnp.float32),
            pltpu.SemaphoreType.DMA,
            pltpu.SemaphoreType.DMA,
            pltpu.SemaphoreType.DMA,
            pltpu.SemaphoreType.DMA,
            pltpu.SemaphoreType.DMA,
            pltpu.SemaphoreType.DMA,
        ],
    )
    return k2(x, table_lin)


# speedup vs baseline: 1.3321x; 1.0657x over previous
import jax
import jax.numpy as jnp
from jax import lax
from jax.experimental import pallas as pl
from jax.experimental.pallas import tpu as pltpu
from jax.experimental.pallas import tpu_sc as plsc

NUM_EMB = 1_000_000
DIM = 32
PAD_IDX = 1
B = 4096
L = 200

NC = 2
NS = 16
NW = NC * NS

BLK = 128
N_FULL_BLK = NUM_EMB // BLK
TAIL = NUM_EMB - N_FULL_BLK * BLK
BLK_PER_W = N_FULL_BLK // NW
N_EXTRA = N_FULL_BLK - BLK_PER_W * NW

ROWS_PER_W = B // NW
CHUNK = 4
N_CHUNKS = ROWS_PER_W // CHUNK
IDX_PER_CHUNK = CHUNK * L
G_SPLIT = (104, 96)
NBUF = 3


def _transpose_block(ibuf, obuf, iota16, ncols):
    for fg in (0, 1):
        f0 = 16 * fg
        frow = f0 + iota16

        @plsc.parallel_loop(0, ncols // 16, unroll=4)
        def cb_body(cb):
            c0 = cb * 16
            c4 = cb * 4
            for d in range(16):
                rot = (d + iota16) & 15
                v = plsc.load_gather(ibuf, [frow, c0 + rot])
                orow = c4 + (rot >> 2)
                ocol = ((rot & 3) << 5) + frow
                plsc.store_scatter(obuf, [orow, ocol], v)


def _k1_body(tT_hbm, out_hbm, ib0, ib1, ob0, ob1, itail, otail,
             rs0, rs1, ws0, ws1):
    wid = lax.axis_index("s") * NC + lax.axis_index("c")
    iota16 = lax.iota(jnp.int32, 16)
    ibufs, obufs = (ib0, ib1), (ob0, ob1)
    rsems, wsems = (rs0, rs1), (ws0, ws1)

    def blk_of(t):
        return wid * BLK_PER_W + t

    def fire_read(t, p):
        col0 = pl.multiple_of(blk_of(t) * BLK, BLK)
        pltpu.async_copy(tT_hbm.at[:, pl.ds(col0, BLK)], ibufs[p], rsems[p])

    def wait_read(p):
        pltpu.make_async_copy(tT_hbm.at[:, pl.ds(0, BLK)], ibufs[p],
                              rsems[p]).wait()

    def fire_write(t, p):
        row0 = pl.multiple_of(blk_of(t) * (BLK // 4), BLK // 4)
        pltpu.async_copy(obufs[p], out_hbm.at[pl.ds(row0, BLK // 4)], wsems[p])

    def wait_write(p):
        pltpu.make_async_copy(obufs[p], out_hbm.at[pl.ds(0, BLK // 4)],
                              wsems[p]).wait()

    fire_read(0, 0)
    fire_read(1, 1)

    def outer(ko, _):
        for p in (0, 1):
            t = 2 * ko + p
            wait_read(p)

            @pl.when(ko > 0)
            def _():
                wait_write(p)
            _transpose_block(ibufs[p], obufs[p], iota16, BLK)

            @pl.when(t + 2 < BLK_PER_W)
            def _():
                fire_read(t + 2, p)
            fire_write(t, p)
        return 0

    lax.fori_loop(0, BLK_PER_W // 2, outer, 0)
    wait_write(0)
    wait_write(1)

    @pl.when(wid < N_EXTRA)
    def _():
        col0 = pl.multiple_of((NW * BLK_PER_W + wid) * BLK, BLK)
        row0 = pl.multiple_of((NW * BLK_PER_W + wid) * (BLK // 4), BLK // 4)
        pltpu.sync_copy(tT_hbm.at[:, pl.ds(col0, BLK)], ibufs[0])
        _transpose_block(ibufs[0], obufs[0], iota16, BLK)
        pltpu.sync_copy(obufs[0], out_hbm.at[pl.ds(row0, BLK // 4)])

    @pl.when(wid == N_EXTRA)
    def _():
        col0 = N_FULL_BLK * BLK
        pltpu.sync_copy(tT_hbm.at[:, pl.ds(col0, TAIL)], itail)
        _transpose_block(itail, otail, iota16, TAIL)
        pltpu.sync_copy(otail, out_hbm.at[pl.ds(N_FULL_BLK * (BLK // 4), TAIL // 4)])


def _k2_body(x_hbm, table_hbm, out_hbm,
             idx0, idx1, idx2, rows0, rows1, rows2, out_v, pad_v,
             is0, is1, is2, rs0, rs1, rs2):
    wid = lax.axis_index("s") * NC + lax.axis_index("c")

    pltpu.sync_copy(table_hbm.at[pl.ds(PAD_IDX, 1)], pad_v)
    t1_lo = pad_v[0, pl.ds(0, 16)]
    t1_hi = pad_v[0, pl.ds(16, 16)]
    inv_l = jnp.float32(1.0 / L)
    tail_mask = lax.iota(jnp.int32, 16) >= 8

    idxs = (idx0, idx1, idx2)
    rows = (rows0, rows1, rows2)
    isems = (is0, is1, is2)
    rsems = (rs0, rs1, rs2)

    def fire_idx(k, p):
        pltpu.async_copy(
            x_hbm.at[pl.ds((wid * N_CHUNKS + k) * CHUNK, CHUNK)],
            idxs[p], isems[p])

    def wait_idx(p):
        pltpu.make_async_copy(
            x_hbm.at[pl.ds(0, CHUNK)], idxs[p], isems[p]).wait()

    def fire_gathers(p):
        for b in range(CHUNK):
            off = 0
            for g in G_SPLIT:
                pltpu.async_copy(
                    table_hbm.at[idxs[p].at[b, pl.ds(off, g)]],
                    rows[p].at[pl.ds(b * L + off, g)],
                    rsems[p],
                )
                off += g

    def drain_rows(p):
        pltpu.make_async_copy(
            table_hbm.at[pl.ds(0, IDX_PER_CHUNK)], rows[p], rsems[p]
        ).wait()

    def accumulate(k, p):
        idx_v, rows_v = idxs[p], rows[p]
        for b in range(CHUNK):
            cnt = jnp.zeros((16,), jnp.float32)
            for t in range(12):
                iv = idx_v[b, pl.ds(t * 16, 16)]
                cnt = cnt + jnp.where(iv == PAD_IDX, 1.0, 0.0)
            ivt = idx_v[b, pl.ds(L - 16, 16)]
            cnt = cnt + jnp.where((ivt == PAD_IDX) & tail_mask, 1.0, 0.0)
            npad = jnp.sum(cnt)

            def row_body(t, accs):
                base = b * L + t * 16
                accs = list(accs)
                for u in range(16):
                    fr = base + u
                    accs[u % 4] = accs[u % 4] + rows_v[fr, pl.ds(0, 16)]
                    accs[4 + u % 4] = accs[4 + u % 4] + rows_v[fr, pl.ds(16, 16)]
                return tuple(accs)

            zero = jnp.zeros((16,), jnp.float32)
            accs = list(lax.fori_loop(0, 12, row_body, (zero,) * 8))
            for u in range(8):
                fr = b * L + 192 + u
                accs[u % 4] = accs[u % 4] + rows_v[fr, pl.ds(0, 16)]
                accs[4 + u % 4] = accs[4 + u % 4] + rows_v[fr, pl.ds(16, 16)]
            s_lo = (accs[0] + accs[1]) + (accs[2] + accs[3])
            s_hi = (accs[4] + accs[5]) + (accs[6] + accs[7])
            row = k * CHUNK + b
            out_v[row, pl.ds(0, 16)] = (s_lo - npad * t1_lo) * inv_l
            out_v[row, pl.ds(16, 16)] = (s_hi - npad * t1_hi) * inv_l

    fire_idx(0, 0)
    fire_idx(1, 1)
    fire_idx(2, 2)
    wait_idx(0)
    fire_gathers(0)
    wait_idx(1)
    fire_gathers(1)

    N_TRIPLES = N_CHUNKS // 3

    def outer(ko, _):
        for j in range(3):
            k = 3 * ko + j
            p = j
            pn2 = (j + 2) % 3
            wait_idx(pn2)
            fire_gathers(pn2)
            drain_rows(p)
            accumulate(k, p)
            if j < 2:
                fire_idx(k + 3, p)
            else:
                @pl.when(ko < N_TRIPLES - 1)
                def _():
                    fire_idx(k + 3, p)
        return 0

    lax.fori_loop(0, N_TRIPLES, outer, 0)
    for k, p in ((N_CHUNKS - 2, 0), (N_CHUNKS - 1, 1)):
        drain_rows(p)
        accumulate(k, p)

    pltpu.sync_copy(out_v, out_hbm.at[pl.ds(wid * ROWS_PER_W, ROWS_PER_W)])


@jax.jit
def kernel(x, table):
    mesh = plsc.VectorSubcoreMesh(
        core_axis_name="c", subcore_axis_name="s", num_cores=NC, num_subcores=NS
    )

    k1 = pl.kernel(
        _k1_body,
        out_type=jax.ShapeDtypeStruct((NUM_EMB // 4, 4 * DIM), jnp.float32),
        mesh=mesh,
        compiler_params=pltpu.CompilerParams(
            needs_layout_passes=False, use_tc_tiling_on_sc=True
        ),
        scratch_types=[
            pltpu.VMEM((DIM, BLK), jnp.float32),
            pltpu.VMEM((DIM, BLK), jnp.float32),
            pltpu.VMEM((BLK // 4, 4 * DIM), jnp.float32),
            pltpu.VMEM((BLK // 4, 4 * DIM), jnp.float32),
            pltpu.VMEM((DIM, TAIL), jnp.float32),
            pltpu.VMEM((TAIL // 4, 4 * DIM), jnp.float32),
            pltpu.SemaphoreType.DMA,
            pltpu.SemaphoreType.DMA,
            pltpu.SemaphoreType.DMA,
            pltpu.SemaphoreType.DMA,
        ],
    )
    table_lin = k1(table.T).reshape(NUM_EMB, DIM)

    k2 = pl.kernel(
        _k2_body,
        out_type=jax.ShapeDtypeStruct((B, DIM), jnp.float32),
        mesh=mesh,
        compiler_params=pltpu.CompilerParams(
            needs_layout_passes=False, use_tc_tiling_on_sc=False
        ),
        scratch_types=[
            pltpu.VMEM((CHUNK, L), jnp.int32),
            pltpu.VMEM((CHUNK, L), jnp.int32),
            pltpu.VMEM((CHUNK, L), jnp.int32),
            pltpu.VMEM((IDX_PER_CHUNK, DIM), jnp.float32),
            pltpu.VMEM((IDX_PER_CHUNK, DIM), jnp.float32),
            pltpu.VMEM((IDX_PER_CHUNK, DIM), jnp.float32),
            pltpu.VMEM((ROWS_PER_W, DIM), jnp.float32),
            pltpu.VMEM((1, DIM), jnp.float32),
            pltpu.SemaphoreType.DMA,
            pltpu.SemaphoreType.DMA,
            pltpu.SemaphoreType.DMA,
            pltpu.SemaphoreType.DMA,
            pltpu.SemaphoreType.DMA,
            pltpu.SemaphoreType.DMA,
        ],
    )
    return k2(x, table_lin)


# speedup vs baseline: 1.5109x; 1.1342x over previous
import jax
import jax.numpy as jnp
from jax import lax
from jax.experimental import pallas as pl
from jax.experimental.pallas import tpu as pltpu
from jax.experimental.pallas import tpu_sc as plsc

NUM_EMB = 1_000_000
DIM = 32
PAD_IDX = 1
B = 4096
L = 200

NC = 2
NS = 16
NW = NC * NS

BLK = 256
N_FULL_BLK = NUM_EMB // BLK
TAIL = NUM_EMB - N_FULL_BLK * BLK
BLK_PER_W = N_FULL_BLK // NW
N_EXTRA = N_FULL_BLK - BLK_PER_W * NW

ROWS_PER_W = B // NW
CHUNK = 4
N_CHUNKS = ROWS_PER_W // CHUNK
IDX_PER_CHUNK = CHUNK * L
G_SPLIT = (104, 96)
NBUF = 3


def _transpose_block(ibuf, obuf, iota16, ncols):
    for fg in (0, 1):
        f0 = 16 * fg
        frow = f0 + iota16

        @plsc.parallel_loop(0, ncols // 16, unroll=4)
        def cb_body(cb):
            c0 = cb * 16
            c4 = cb * 4
            for d in range(16):
                rot = (d + iota16) & 15
                v = plsc.load_gather(ibuf, [frow, c0 + rot])
                orow = c4 + (rot >> 2)
                ocol = ((rot & 3) << 5) + frow
                plsc.store_scatter(obuf, [orow, ocol], v)


def _k1_body(tT_hbm, out_hbm, ib0, ib1, ob0, ob1, itail, otail,
             rs0, rs1, ws0, ws1):
    wid = lax.axis_index("s") * NC + lax.axis_index("c")
    iota16 = lax.iota(jnp.int32, 16)
    ibufs, obufs = (ib0, ib1), (ob0, ob1)
    rsems, wsems = (rs0, rs1), (ws0, ws1)

    def blk_of(t):
        return wid * BLK_PER_W + t

    def fire_read(t, p):
        col0 = pl.multiple_of(blk_of(t) * BLK, BLK)
        pltpu.async_copy(tT_hbm.at[:, pl.ds(col0, BLK)], ibufs[p], rsems[p])

    def wait_read(p):
        pltpu.make_async_copy(tT_hbm.at[:, pl.ds(0, BLK)], ibufs[p],
                              rsems[p]).wait()

    def fire_write(t, p):
        row0 = pl.multiple_of(blk_of(t) * (BLK // 4), BLK // 4)
        pltpu.async_copy(obufs[p], out_hbm.at[pl.ds(row0, BLK // 4)], wsems[p])

    def wait_write(p):
        pltpu.make_async_copy(obufs[p], out_hbm.at[pl.ds(0, BLK // 4)],
                              wsems[p]).wait()

    fire_read(0, 0)
    fire_read(1, 1)

    def outer(ko, _):
        for p in (0, 1):
            t = 2 * ko + p
            wait_read(p)

            @pl.when(ko > 0)
            def _():
                wait_write(p)
            _transpose_block(ibufs[p], obufs[p], iota16, BLK)

            @pl.when(t + 2 < BLK_PER_W)
            def _():
                fire_read(t + 2, p)
            fire_write(t, p)
        return 0

    lax.fori_loop(0, BLK_PER_W // 2, outer, 0)
    wait_write(0)
    wait_write(1)

    @pl.when(wid < N_EXTRA)
    def _():
        col0 = pl.multiple_of((NW * BLK_PER_W + wid) * BLK, BLK)
        row0 = pl.multiple_of((NW * BLK_PER_W + wid) * (BLK // 4), BLK // 4)
        pltpu.sync_copy(tT_hbm.at[:, pl.ds(col0, BLK)], ibufs[0])
        _transpose_block(ibufs[0], obufs[0], iota16, BLK)
        pltpu.sync_copy(obufs[0], out_hbm.at[pl.ds(row0, BLK // 4)])

    @pl.when(wid == N_EXTRA)
    def _():
        col0 = N_FULL_BLK * BLK
        pltpu.sync_copy(tT_hbm.at[:, pl.ds(col0, TAIL)], itail)
        _transpose_block(itail, otail, iota16, TAIL)
        pltpu.sync_copy(otail, out_hbm.at[pl.ds(N_FULL_BLK * (BLK // 4), TAIL // 4)])


def _k2_body(x_hbm, table_hbm, out_hbm,
             idx0, idx1, idx2, rows0, rows1, rows2, out_v, pad_v,
             is0, is1, is2, rs0, rs1, rs2):
    wid = lax.axis_index("s") * NC + lax.axis_index("c")

    pltpu.sync_copy(table_hbm.at[pl.ds(PAD_IDX, 1)], pad_v)
    t1_lo = pad_v[0, pl.ds(0, 16)]
    t1_hi = pad_v[0, pl.ds(16, 16)]
    inv_l = jnp.float32(1.0 / L)
    tail_mask = lax.iota(jnp.int32, 16) >= 8

    idxs = (idx0, idx1, idx2)
    rows = (rows0, rows1, rows2)
    isems = (is0, is1, is2)
    rsems = (rs0, rs1, rs2)

    def fire_idx(k, p):
        pltpu.async_copy(
            x_hbm.at[pl.ds((wid * N_CHUNKS + k) * CHUNK, CHUNK)],
            idxs[p], isems[p])

    def wait_idx(p):
        pltpu.make_async_copy(
            x_hbm.at[pl.ds(0, CHUNK)], idxs[p], isems[p]).wait()

    def fire_gathers(p):
        for b in range(CHUNK):
            off = 0
            for g in G_SPLIT:
                pltpu.async_copy(
                    table_hbm.at[idxs[p].at[b, pl.ds(off, g)]],
                    rows[p].at[pl.ds(b * L + off, g)],
                    rsems[p],
                )
                off += g

    def drain_rows(p):
        pltpu.make_async_copy(
            table_hbm.at[pl.ds(0, IDX_PER_CHUNK)], rows[p], rsems[p]
        ).wait()

    def accumulate(k, p):
        idx_v, rows_v = idxs[p], rows[p]
        for b in range(CHUNK):
            cnt = jnp.zeros((16,), jnp.float32)
            for t in range(12):
                iv = idx_v[b, pl.ds(t * 16, 16)]
                cnt = cnt + jnp.where(iv == PAD_IDX, 1.0, 0.0)
            ivt = idx_v[b, pl.ds(L - 16, 16)]
            cnt = cnt + jnp.where((ivt == PAD_IDX) & tail_mask, 1.0, 0.0)
            npad = jnp.sum(cnt)

            def row_body(t, accs):
                base = b * L + t * 16
                accs = list(accs)
                for u in range(16):
                    fr = base + u
                    accs[u % 4] = accs[u % 4] + rows_v[fr, pl.ds(0, 16)]
                    accs[4 + u % 4] = accs[4 + u % 4] + rows_v[fr, pl.ds(16, 16)]
                return tuple(accs)

            zero = jnp.zeros((16,), jnp.float32)
            accs = list(lax.fori_loop(0, 12, row_body, (zero,) * 8))
            for u in range(8):
                fr = b * L + 192 + u
                accs[u % 4] = accs[u % 4] + rows_v[fr, pl.ds(0, 16)]
                accs[4 + u % 4] = accs[4 + u % 4] + rows_v[fr, pl.ds(16, 16)]
            s_lo = (accs[0] + accs[1]) + (accs[2] + accs[3])
            s_hi = (accs[4] + accs[5]) + (accs[6] + accs[7])
            row = k * CHUNK + b
            out_v[row, pl.ds(0, 16)] = (s_lo - npad * t1_lo) * inv_l
            out_v[row, pl.ds(16, 16)] = (s_hi - npad * t1_hi) * inv_l

    fire_idx(0, 0)
    fire_idx(1, 1)
    fire_idx(2, 2)
    wait_idx(0)
    fire_gathers(0)
    wait_idx(1)
    fire_gathers(1)

    N_TRIPLES = N_CHUNKS // 3

    def outer(ko, _):
        for j in range(3):
            k = 3 * ko + j
            p = j
            pn2 = (j + 2) % 3
            wait_idx(pn2)
            fire_gathers(pn2)
            drain_rows(p)
            accumulate(k, p)
            if j < 2:
                fire_idx(k + 3, p)
            else:
                @pl.when(ko < N_TRIPLES - 1)
                def _():
                    fire_idx(k + 3, p)
        return 0

    lax.fori_loop(0, N_TRIPLES, outer, 0)
    for k, p in ((N_CHUNKS - 2, 0), (N_CHUNKS - 1, 1)):
        drain_rows(p)
        accumulate(k, p)

    pltpu.sync_copy(out_v, out_hbm.at[pl.ds(wid * ROWS_PER_W, ROWS_PER_W)])


@jax.jit
def kernel(x, table):
    mesh = plsc.VectorSubcoreMesh(
        core_axis_name="c", subcore_axis_name="s", num_cores=NC, num_subcores=NS
    )

    k1 = pl.kernel(
        _k1_body,
        out_type=jax.ShapeDtypeStruct((NUM_EMB // 4, 4 * DIM), jnp.float32),
        mesh=mesh,
        compiler_params=pltpu.CompilerParams(
            needs_layout_passes=False, use_tc_tiling_on_sc=True
        ),
        scratch_types=[
            pltpu.VMEM((DIM, BLK), jnp.float32),
            pltpu.VMEM((DIM, BLK), jnp.float32),
            pltpu.VMEM((BLK // 4, 4 * DIM), jnp.float32),
            pltpu.VMEM((BLK // 4, 4 * DIM), jnp.float32),
            pltpu.VMEM((DIM, TAIL), jnp.float32),
            pltpu.VMEM((TAIL // 4, 4 * DIM), jnp.float32),
            pltpu.SemaphoreType.DMA,
            pltpu.SemaphoreType.DMA,
            pltpu.SemaphoreType.DMA,
            pltpu.SemaphoreType.DMA,
        ],
    )
    table_lin = k1(table.T).reshape(NUM_EMB, DIM)

    k2 = pl.kernel(
        _k2_body,
        out_type=jax.ShapeDtypeStruct((B, DIM), jnp.float32),
        mesh=mesh,
        compiler_params=pltpu.CompilerParams(
            needs_layout_passes=False, use_tc_tiling_on_sc=False
        ),
        scratch_types=[
            pltpu.VMEM((CHUNK, L), jnp.int32),
            pltpu.VMEM((CHUNK, L), jnp.int32),
            pltpu.VMEM((CHUNK, L), jnp.int32),
            pltpu.VMEM((IDX_PER_CHUNK, DIM), jnp.float32),
            pltpu.VMEM((IDX_PER_CHUNK, DIM), jnp.float32),
            pltpu.VMEM((IDX_PER_CHUNK, DIM), jnp.float32),
            pltpu.VMEM((ROWS_PER_W, DIM), jnp.float32),
            pltpu.VMEM((1, DIM), jnp.float32),
            pltpu.SemaphoreType.DMA,
            pltpu.SemaphoreType.DMA,
            pltpu.SemaphoreType.DMA,
            pltpu.SemaphoreType.DMA,
            pltpu.SemaphoreType.DMA,
            pltpu.SemaphoreType.DMA,
        ],
    )
    return k2(x, table_lin)


# speedup vs baseline: 1.6928x; 1.1204x over previous
import jax
import jax.numpy as jnp
from jax import lax
from jax.experimental import pallas as pl
from jax.experimental.pallas import tpu as pltpu
from jax.experimental.pallas import tpu_sc as plsc

NUM_EMB = 1_000_000
DIM = 32
PAD_IDX = 1
B = 4096
L = 200

NC = 2
NS = 16
NW = NC * NS

BLK = 512
N_FULL_BLK = NUM_EMB // BLK
TAIL = NUM_EMB - N_FULL_BLK * BLK
BLK_PER_W = N_FULL_BLK // NW
N_EXTRA = N_FULL_BLK - BLK_PER_W * NW

ROWS_PER_W = B // NW
CHUNK = 4
N_CHUNKS = ROWS_PER_W // CHUNK
IDX_PER_CHUNK = CHUNK * L
G_SPLIT = (104, 96)
NBUF = 3


def _transpose_block(ibuf, obuf, iota16, ncols):
    for fg in (0, 1):
        f0 = 16 * fg
        frow = f0 + iota16

        @plsc.parallel_loop(0, ncols // 16, unroll=4)
        def cb_body(cb):
            c0 = cb * 16
            c4 = cb * 4
            for d in range(16):
                rot = (d + iota16) & 15
                v = plsc.load_gather(ibuf, [frow, c0 + rot])
                orow = c4 + (rot >> 2)
                ocol = ((rot & 3) << 5) + frow
                plsc.store_scatter(obuf, [orow, ocol], v)


def _k1_body(tT_hbm, out_hbm, ib0, ib1, ob0, ob1, itail, otail,
             rs0, rs1, ws0, ws1):
    wid = lax.axis_index("s") * NC + lax.axis_index("c")
    iota16 = lax.iota(jnp.int32, 16)
    ibufs, obufs = (ib0, ib1), (ob0, ob1)
    rsems, wsems = (rs0, rs1), (ws0, ws1)

    def blk_of(t):
        return wid * BLK_PER_W + t

    def fire_read(t, p):
        col0 = pl.multiple_of(blk_of(t) * BLK, BLK)
        pltpu.async_copy(tT_hbm.at[:, pl.ds(col0, BLK)], ibufs[p], rsems[p])

    def wait_read(p):
        pltpu.make_async_copy(tT_hbm.at[:, pl.ds(0, BLK)], ibufs[p],
                              rsems[p]).wait()

    def fire_write(t, p):
        row0 = pl.multiple_of(blk_of(t) * (BLK // 4), BLK // 4)
        pltpu.async_copy(obufs[p], out_hbm.at[pl.ds(row0, BLK // 4)], wsems[p])

    def wait_write(p):
        pltpu.make_async_copy(obufs[p], out_hbm.at[pl.ds(0, BLK // 4)],
                              wsems[p]).wait()

    fire_read(0, 0)
    fire_read(1, 1)

    def outer(ko, _):
        for p in (0, 1):
            t = 2 * ko + p
            wait_read(p)

            @pl.when(ko > 0)
            def _():
                wait_write(p)
            _transpose_block(ibufs[p], obufs[p], iota16, BLK)

            @pl.when(t + 2 < BLK_PER_W)
            def _():
                fire_read(t + 2, p)
            fire_write(t, p)
        return 0

    lax.fori_loop(0, BLK_PER_W // 2, outer, 0)
    if BLK_PER_W % 2:
        t = BLK_PER_W - 1
        wait_read(0)
        wait_write(0)
        _transpose_block(ibufs[0], obufs[0], iota16, BLK)
        fire_write(t, 0)
    wait_write(0)
    wait_write(1)

    @pl.when(wid < N_EXTRA)
    def _():
        col0 = pl.multiple_of((NW * BLK_PER_W + wid) * BLK, BLK)
        row0 = pl.multiple_of((NW * BLK_PER_W + wid) * (BLK // 4), BLK // 4)
        pltpu.sync_copy(tT_hbm.at[:, pl.ds(col0, BLK)], ibufs[0])
        _transpose_block(ibufs[0], obufs[0], iota16, BLK)
        pltpu.sync_copy(obufs[0], out_hbm.at[pl.ds(row0, BLK // 4)])

    @pl.when(wid == N_EXTRA)
    def _():
        col0 = N_FULL_BLK * BLK
        pltpu.sync_copy(tT_hbm.at[:, pl.ds(col0, TAIL)], itail)
        _transpose_block(itail, otail, iota16, TAIL)
        pltpu.sync_copy(otail, out_hbm.at[pl.ds(N_FULL_BLK * (BLK // 4), TAIL // 4)])


def _k2_body(x_hbm, table_hbm, out_hbm,
             idx0, idx1, idx2, rows0, rows1, rows2, out_v, pad_v,
             is0, is1, is2, rs0, rs1, rs2):
    wid = lax.axis_index("s") * NC + lax.axis_index("c")

    pltpu.sync_copy(table_hbm.at[pl.ds(PAD_IDX, 1)], pad_v)
    t1_lo = pad_v[0, pl.ds(0, 16)]
    t1_hi = pad_v[0, pl.ds(16, 16)]
    inv_l = jnp.float32(1.0 / L)
    tail_mask = lax.iota(jnp.int32, 16) >= 8

    idxs = (idx0, idx1, idx2)
    rows = (rows0, rows1, rows2)
    isems = (is0, is1, is2)
    rsems = (rs0, rs1, rs2)

    def fire_idx(k, p):
        pltpu.async_copy(
            x_hbm.at[pl.ds((wid * N_CHUNKS + k) * CHUNK, CHUNK)],
            idxs[p], isems[p])

    def wait_idx(p):
        pltpu.make_async_copy(
            x_hbm.at[pl.ds(0, CHUNK)], idxs[p], isems[p]).wait()

    def fire_gathers(p):
        for b in range(CHUNK):
            off = 0
            for g in G_SPLIT:
                pltpu.async_copy(
                    table_hbm.at[idxs[p].at[b, pl.ds(off, g)]],
                    rows[p].at[pl.ds(b * L + off, g)],
                    rsems[p],
                )
                off += g

    def drain_rows(p):
        pltpu.make_async_copy(
            table_hbm.at[pl.ds(0, IDX_PER_CHUNK)], rows[p], rsems[p]
        ).wait()

    def accumulate(k, p):
        idx_v, rows_v = idxs[p], rows[p]
        for b in range(CHUNK):
            cnt = jnp.zeros((16,), jnp.float32)
            for t in range(12):
                iv = idx_v[b, pl.ds(t * 16, 16)]
                cnt = cnt + jnp.where(iv == PAD_IDX, 1.0, 0.0)
            ivt = idx_v[b, pl.ds(L - 16, 16)]
            cnt = cnt + jnp.where((ivt == PAD_IDX) & tail_mask, 1.0, 0.0)
            npad = jnp.sum(cnt)

            def row_body(t, accs):
                base = b * L + t * 16
                accs = list(accs)
                for u in range(16):
                    fr = base + u
                    accs[u % 4] = accs[u % 4] + rows_v[fr, pl.ds(0, 16)]
                    accs[4 + u % 4] = accs[4 + u % 4] + rows_v[fr, pl.ds(16, 16)]
                return tuple(accs)

            zero = jnp.zeros((16,), jnp.float32)
            accs = list(lax.fori_loop(0, 12, row_body, (zero,) * 8))
            for u in range(8):
                fr = b * L + 192 + u
                accs[u % 4] = accs[u % 4] + rows_v[fr, pl.ds(0, 16)]
                accs[4 + u % 4] = accs[4 + u % 4] + rows_v[fr, pl.ds(16, 16)]
            s_lo = (accs[0] + accs[1]) + (accs[2] + accs[3])
            s_hi = (accs[4] + accs[5]) + (accs[6] + accs[7])
            row = k * CHUNK + b
            out_v[row, pl.ds(0, 16)] = (s_lo - npad * t1_lo) * inv_l
            out_v[row, pl.ds(16, 16)] = (s_hi - npad * t1_hi) * inv_l

    fire_idx(0, 0)
    fire_idx(1, 1)
    fire_idx(2, 2)
    wait_idx(0)
    fire_gathers(0)
    wait_idx(1)
    fire_gathers(1)

    N_TRIPLES = N_CHUNKS // 3

    def outer(ko, _):
        for j in range(3):
            k = 3 * ko + j
            p = j
            pn2 = (j + 2) % 3
            wait_idx(pn2)
            fire_gathers(pn2)
            drain_rows(p)
            accumulate(k, p)
            if j < 2:
                fire_idx(k + 3, p)
            else:
                @pl.when(ko < N_TRIPLES - 1)
                def _():
                    fire_idx(k + 3, p)
        return 0

    lax.fori_loop(0, N_TRIPLES, outer, 0)
    for k, p in ((N_CHUNKS - 2, 0), (N_CHUNKS - 1, 1)):
        drain_rows(p)
        accumulate(k, p)

    pltpu.sync_copy(out_v, out_hbm.at[pl.ds(wid * ROWS_PER_W, ROWS_PER_W)])


@jax.jit
def kernel(x, table):
    mesh = plsc.VectorSubcoreMesh(
        core_axis_name="c", subcore_axis_name="s", num_cores=NC, num_subcores=NS
    )

    k1 = pl.kernel(
        _k1_body,
        out_type=jax.ShapeDtypeStruct((NUM_EMB // 4, 4 * DIM), jnp.float32),
        mesh=mesh,
        compiler_params=pltpu.CompilerParams(
            needs_layout_passes=False, use_tc_tiling_on_sc=True
        ),
        scratch_types=[
            pltpu.VMEM((DIM, BLK), jnp.float32),
            pltpu.VMEM((DIM, BLK), jnp.float32),
            pltpu.VMEM((BLK // 4, 4 * DIM), jnp.float32),
            pltpu.VMEM((BLK // 4, 4 * DIM), jnp.float32),
            pltpu.VMEM((DIM, TAIL), jnp.float32),
            pltpu.VMEM((TAIL // 4, 4 * DIM), jnp.float32),
            pltpu.SemaphoreType.DMA,
            pltpu.SemaphoreType.DMA,
            pltpu.SemaphoreType.DMA,
            pltpu.SemaphoreType.DMA,
        ],
    )
    table_lin = k1(table.T).reshape(NUM_EMB, DIM)

    k2 = pl.kernel(
        _k2_body,
        out_type=jax.ShapeDtypeStruct((B, DIM), jnp.float32),
        mesh=mesh,
        compiler_params=pltpu.CompilerParams(
            needs_layout_passes=False, use_tc_tiling_on_sc=False
        ),
        scratch_types=[
            pltpu.VMEM((CHUNK, L), jnp.int32),
            pltpu.VMEM((CHUNK, L), jnp.int32),
            pltpu.VMEM((CHUNK, L), jnp.int32),
            pltpu.VMEM((IDX_PER_CHUNK, DIM), jnp.float32),
            pltpu.VMEM((IDX_PER_CHUNK, DIM), jnp.float32),
            pltpu.VMEM((IDX_PER_CHUNK, DIM), jnp.float32),
            pltpu.VMEM((ROWS_PER_W, DIM), jnp.float32),
            pltpu.VMEM((1, DIM), jnp.float32),
            pltpu.SemaphoreType.DMA,
            pltpu.SemaphoreType.DMA,
            pltpu.SemaphoreType.DMA,
            pltpu.SemaphoreType.DMA,
            pltpu.SemaphoreType.DMA,
            pltpu.SemaphoreType.DMA,
        ],
    )
    return k2(x, table_lin)


# speedup vs baseline: 1.7844x; 1.0541x over previous
import jax
import jax.numpy as jnp
from jax import lax
from jax.experimental import pallas as pl
from jax.experimental.pallas import tpu as pltpu
from jax.experimental.pallas import tpu_sc as plsc

NUM_EMB = 1_000_000
DIM = 32
PAD_IDX = 1
B = 4096
L = 200

NC = 2
NS = 16
NW = NC * NS

BLK = 768
N_FULL_BLK = NUM_EMB // BLK
TAIL = NUM_EMB - N_FULL_BLK * BLK
BLK_PER_W = N_FULL_BLK // NW
N_EXTRA = N_FULL_BLK - BLK_PER_W * NW

ROWS_PER_W = B // NW
CHUNK = 4
N_CHUNKS = ROWS_PER_W // CHUNK
IDX_PER_CHUNK = CHUNK * L
G_SPLIT = (104, 96)
NBUF = 3


def _transpose_block(ibuf, obuf, iota16, ncols):
    for fg in (0, 1):
        f0 = 16 * fg
        frow = f0 + iota16

        @plsc.parallel_loop(0, ncols // 16, unroll=4)
        def cb_body(cb):
            c0 = cb * 16
            c4 = cb * 4
            for d in range(16):
                rot = (d + iota16) & 15
                v = plsc.load_gather(ibuf, [frow, c0 + rot])
                orow = c4 + (rot >> 2)
                ocol = ((rot & 3) << 5) + frow
                plsc.store_scatter(obuf, [orow, ocol], v)


def _k1_body(tT_hbm, out_hbm, ib0, ib1, ob0, ob1, itail, otail,
             rs0, rs1, ws0, ws1):
    wid = lax.axis_index("s") * NC + lax.axis_index("c")
    iota16 = lax.iota(jnp.int32, 16)
    ibufs, obufs = (ib0, ib1), (ob0, ob1)
    rsems, wsems = (rs0, rs1), (ws0, ws1)

    def blk_of(t):
        return wid * BLK_PER_W + t

    def fire_read(t, p):
        col0 = pl.multiple_of(blk_of(t) * BLK, BLK)
        pltpu.async_copy(tT_hbm.at[:, pl.ds(col0, BLK)], ibufs[p], rsems[p])

    def wait_read(p):
        pltpu.make_async_copy(tT_hbm.at[:, pl.ds(0, BLK)], ibufs[p],
                              rsems[p]).wait()

    def fire_write(t, p):
        row0 = pl.multiple_of(blk_of(t) * (BLK // 4), BLK // 4)
        pltpu.async_copy(obufs[p], out_hbm.at[pl.ds(row0, BLK // 4)], wsems[p])

    def wait_write(p):
        pltpu.make_async_copy(obufs[p], out_hbm.at[pl.ds(0, BLK // 4)],
                              wsems[p]).wait()

    fire_read(0, 0)
    fire_read(1, 1)

    def outer(ko, _):
        for p in (0, 1):
            t = 2 * ko + p
            wait_read(p)

            @pl.when(ko > 0)
            def _():
                wait_write(p)
            _transpose_block(ibufs[p], obufs[p], iota16, BLK)

            @pl.when(t + 2 < BLK_PER_W)
            def _():
                fire_read(t + 2, p)
            fire_write(t, p)
        return 0

    lax.fori_loop(0, BLK_PER_W // 2, outer, 0)
    if BLK_PER_W % 2:
        t = BLK_PER_W - 1
        wait_read(0)
        wait_write(0)
        _transpose_block(ibufs[0], obufs[0], iota16, BLK)
        fire_write(t, 0)
    wait_write(0)
    wait_write(1)

    @pl.when(wid < N_EXTRA)
    def _():
        col0 = pl.multiple_of((NW * BLK_PER_W + wid) * BLK, BLK)
        row0 = pl.multiple_of((NW * BLK_PER_W + wid) * (BLK // 4), BLK // 4)
        pltpu.sync_copy(tT_hbm.at[:, pl.ds(col0, BLK)], ibufs[0])
        _transpose_block(ibufs[0], obufs[0], iota16, BLK)
        pltpu.sync_copy(obufs[0], out_hbm.at[pl.ds(row0, BLK // 4)])

    @pl.when(wid == N_EXTRA)
    def _():
        col0 = N_FULL_BLK * BLK
        pltpu.sync_copy(tT_hbm.at[:, pl.ds(col0, TAIL)], itail)
        _transpose_block(itail, otail, iota16, TAIL)
        pltpu.sync_copy(otail, out_hbm.at[pl.ds(N_FULL_BLK * (BLK // 4), TAIL // 4)])


def _k2_body(x_hbm, table_hbm, out_hbm,
             idx0, idx1, idx2, rows0, rows1, rows2, out_v, pad_v,
             is0, is1, is2, rs0, rs1, rs2):
    wid = lax.axis_index("s") * NC + lax.axis_index("c")

    pltpu.sync_copy(table_hbm.at[pl.ds(PAD_IDX, 1)], pad_v)
    t1_lo = pad_v[0, pl.ds(0, 16)]
    t1_hi = pad_v[0, pl.ds(16, 16)]
    inv_l = jnp.float32(1.0 / L)
    tail_mask = lax.iota(jnp.int32, 16) >= 8

    idxs = (idx0, idx1, idx2)
    rows = (rows0, rows1, rows2)
    isems = (is0, is1, is2)
    rsems = (rs0, rs1, rs2)

    def fire_idx(k, p):
        pltpu.async_copy(
            x_hbm.at[pl.ds((wid * N_CHUNKS + k) * CHUNK, CHUNK)],
            idxs[p], isems[p])

    def wait_idx(p):
        pltpu.make_async_copy(
            x_hbm.at[pl.ds(0, CHUNK)], idxs[p], isems[p]).wait()

    def fire_gathers(p):
        for b in range(CHUNK):
            off = 0
            for g in G_SPLIT:
                pltpu.async_copy(
                    table_hbm.at[idxs[p].at[b, pl.ds(off, g)]],
                    rows[p].at[pl.ds(b * L + off, g)],
                    rsems[p],
                )
                off += g

    def drain_rows(p):
        pltpu.make_async_copy(
            table_hbm.at[pl.ds(0, IDX_PER_CHUNK)], rows[p], rsems[p]
        ).wait()

    def accumulate(k, p):
        idx_v, rows_v = idxs[p], rows[p]
        for b in range(CHUNK):
            cnt = jnp.zeros((16,), jnp.float32)
            for t in range(12):
                iv = idx_v[b, pl.ds(t * 16, 16)]
                cnt = cnt + jnp.where(iv == PAD_IDX, 1.0, 0.0)
            ivt = idx_v[b, pl.ds(L - 16, 16)]
            cnt = cnt + jnp.where((ivt == PAD_IDX) & tail_mask, 1.0, 0.0)
            npad = jnp.sum(cnt)

            def row_body(t, accs):
                base = b * L + t * 16
                accs = list(accs)
                for u in range(16):
                    fr = base + u
                    accs[u % 4] = accs[u % 4] + rows_v[fr, pl.ds(0, 16)]
                    accs[4 + u % 4] = accs[4 + u % 4] + rows_v[fr, pl.ds(16, 16)]
                return tuple(accs)

            zero = jnp.zeros((16,), jnp.float32)
            accs = list(lax.fori_loop(0, 12, row_body, (zero,) * 8))
            for u in range(8):
                fr = b * L + 192 + u
                accs[u % 4] = accs[u % 4] + rows_v[fr, pl.ds(0, 16)]
                accs[4 + u % 4] = accs[4 + u % 4] + rows_v[fr, pl.ds(16, 16)]
            s_lo = (accs[0] + accs[1]) + (accs[2] + accs[3])
            s_hi = (accs[4] + accs[5]) + (accs[6] + accs[7])
            row = k * CHUNK + b
            out_v[row, pl.ds(0, 16)] = (s_lo - npad * t1_lo) * inv_l
            out_v[row, pl.ds(16, 16)] = (s_hi - npad * t1_hi) * inv_l

    fire_idx(0, 0)
    fire_idx(1, 1)
    fire_idx(2, 2)
    wait_idx(0)
    fire_gathers(0)
    wait_idx(1)
    fire_gathers(1)

    N_TRIPLES = N_CHUNKS // 3

    def outer(ko, _):
        for j in range(3):
            k = 3 * ko + j
            p = j
            pn2 = (j + 2) % 3
            wait_idx(pn2)
            fire_gathers(pn2)
            drain_rows(p)
            accumulate(k, p)
            if j < 2:
                fire_idx(k + 3, p)
            else:
                @pl.when(ko < N_TRIPLES - 1)
                def _():
                    fire_idx(k + 3, p)
        return 0

    lax.fori_loop(0, N_TRIPLES, outer, 0)
    for k, p in ((N_CHUNKS - 2, 0), (N_CHUNKS - 1, 1)):
        drain_rows(p)
        accumulate(k, p)

    pltpu.sync_copy(out_v, out_hbm.at[pl.ds(wid * ROWS_PER_W, ROWS_PER_W)])


@jax.jit
def kernel(x, table):
    mesh = plsc.VectorSubcoreMesh(
        core_axis_name="c", subcore_axis_name="s", num_cores=NC, num_subcores=NS
    )

    k1 = pl.kernel(
        _k1_body,
        out_type=jax.ShapeDtypeStruct((NUM_EMB // 4, 4 * DIM), jnp.float32),
        mesh=mesh,
        compiler_params=pltpu.CompilerParams(
            needs_layout_passes=False, use_tc_tiling_on_sc=True
        ),
        scratch_types=[
            pltpu.VMEM((DIM, BLK), jnp.float32),
            pltpu.VMEM((DIM, BLK), jnp.float32),
            pltpu.VMEM((BLK // 4, 4 * DIM), jnp.float32),
            pltpu.VMEM((BLK // 4, 4 * DIM), jnp.float32),
            pltpu.VMEM((DIM, TAIL), jnp.float32),
            pltpu.VMEM((TAIL // 4, 4 * DIM), jnp.float32),
            pltpu.SemaphoreType.DMA,
            pltpu.SemaphoreType.DMA,
            pltpu.SemaphoreType.DMA,
            pltpu.SemaphoreType.DMA,
        ],
    )
    table_lin = k1(table.T).reshape(NUM_EMB, DIM)

    k2 = pl.kernel(
        _k2_body,
        out_type=jax.ShapeDtypeStruct((B, DIM), jnp.float32),
        mesh=mesh,
        compiler_params=pltpu.CompilerParams(
            needs_layout_passes=False, use_tc_tiling_on_sc=False
        ),
        scratch_types=[
            pltpu.VMEM((CHUNK, L), jnp.int32),
            pltpu.VMEM((CHUNK, L), jnp.int32),
            pltpu.VMEM((CHUNK, L), jnp.int32),
            pltpu.VMEM((IDX_PER_CHUNK, DIM), jnp.float32),
            pltpu.VMEM((IDX_PER_CHUNK, DIM), jnp.float32),
            pltpu.VMEM((IDX_PER_CHUNK, DIM), jnp.float32),
            pltpu.VMEM((ROWS_PER_W, DIM), jnp.float32),
            pltpu.VMEM((1, DIM), jnp.float32),
            pltpu.SemaphoreType.DMA,
            pltpu.SemaphoreType.DMA,
            pltpu.SemaphoreType.DMA,
            pltpu.SemaphoreType.DMA,
            pltpu.SemaphoreType.DMA,
            pltpu.SemaphoreType.DMA,
        ],
    )
    return k2(x, table_lin)
